# baseline (speedup 1.0000x reference)
import functools
import math

import numpy as np
import jax
import jax.numpy as jnp
from jax import lax
from jax.experimental import pallas as pl
from jax.experimental.pallas import tpu as pltpu
from jax.experimental.pallas import tpu_sc as plsc

SEQ = 2048
HID = 1024
BATCH = 4
ROWS = BATCH * SEQ


def _pe_table():
    position = np.arange(0, SEQ, dtype=np.float32)[:, None]
    div_term = np.exp(
        np.arange(0, HID, 2, dtype=np.float32) * (-math.log(10000.0) / HID)
    )
    pe = np.zeros((SEQ, HID), dtype=np.float32)
    pe[:, 0::2] = np.sin(position * div_term)
    pe[:, 1::2] = np.cos(position * div_term)
    return pe


_PE = _pe_table()

_INFO = plsc.get_sparse_core_info()
NC, NS, LANES = _INFO.num_cores, _INFO.num_subcores, _INFO.num_lanes
NW = NC * NS
RPW = ROWS // NW
CHUNK = 32
NCHUNK = RPW // CHUNK

_mesh = plsc.VectorSubcoreMesh(core_axis_name="c", subcore_axis_name="s")


@functools.partial(
    pl.kernel,
    mesh=_mesh,
    out_type=jax.ShapeDtypeStruct((ROWS, HID), jnp.float32),
    scratch_types=[
        pltpu.VMEM((CHUNK,), jnp.int32),
        pltpu.VMEM((CHUNK, HID), jnp.float32),
        pltpu.VMEM((CHUNK, HID), jnp.float32),
        pltpu.SemaphoreType.DMA,
    ],
)
def _emb(table_hbm, idx_hbm, pe_hbm, out_hbm, idx_v, rows_v, pe_v, sem):
    wid = lax.axis_index("s") * NC + lax.axis_index("c")
    base = wid * RPW

    def chunk_body(c, carry):
        r0 = base + c * CHUNK
        pltpu.sync_copy(idx_hbm.at[pl.ds(r0, CHUNK)], idx_v)
        gather = pltpu.async_copy(table_hbm.at[idx_v], rows_v, sem)
        l0 = lax.rem(r0, SEQ)
        pltpu.sync_copy(pe_hbm.at[pl.ds(l0, CHUNK)], pe_v)
        gather.wait()

        def row_body(i, _):
            def vec_body(j, __):
                sl = pl.ds(j * LANES, LANES)
                rows_v[i, sl] = rows_v[i, sl] + pe_v[i, sl]
                return __

            return lax.fori_loop(0, HID // LANES, vec_body, _)

        lax.fori_loop(0, CHUNK, row_body, 0)
        pltpu.sync_copy(rows_v, out_hbm.at[pl.ds(r0, CHUNK)])
        return carry

    lax.fori_loop(0, NCHUNK, chunk_body, 0)


def kernel(x, table):
    out = _emb(table, x.reshape(-1), jnp.asarray(_PE))
    return out.reshape(BATCH, SEQ, HID)


# speedup vs baseline: 1.9000x; 1.9000x over previous
import functools
import math

import numpy as np
import jax
import jax.numpy as jnp
from jax import lax
from jax.experimental import pallas as pl
from jax.experimental.pallas import tpu as pltpu
from jax.experimental.pallas import tpu_sc as plsc

SEQ = 2048
HID = 1024
BATCH = 4
ROWS = BATCH * SEQ


def _pe_table():
    position = np.arange(0, SEQ, dtype=np.float32)[:, None]
    div_term = np.exp(
        np.arange(0, HID, 2, dtype=np.float32) * (-math.log(10000.0) / HID)
    )
    pe = np.zeros((SEQ, HID), dtype=np.float32)
    pe[:, 0::2] = np.sin(position * div_term)
    pe[:, 1::2] = np.cos(position * div_term)
    return pe


_PE = _pe_table()

_INFO = plsc.get_sparse_core_info()
NC, NS, LANES = _INFO.num_cores, _INFO.num_subcores, _INFO.num_lanes
NW = NC * NS
LPW = SEQ // NW
RPW = BATCH * LPW
CHUNK = 16
QPB = LPW // CHUNK
NBUF = 3
VECS = HID // LANES

_mesh = plsc.VectorSubcoreMesh(core_axis_name="c", subcore_axis_name="s")


@functools.partial(
    pl.kernel,
    mesh=_mesh,
    out_type=jax.ShapeDtypeStruct((ROWS, HID), jnp.float32),
    scratch_types=[
        pltpu.VMEM((RPW,), jnp.int32),
        pltpu.VMEM((LPW, HID), jnp.float32),
    ]
    + [pltpu.VMEM((CHUNK, HID), jnp.float32) for _ in range(NBUF)]
    + [pltpu.SemaphoreType.DMA for _ in range(2 * NBUF + 1)],
)
def _emb(table_hbm, idx_hbm, pe_hbm, out_hbm, idx_v, pe_v, *bufs_and_sems):
    rows = bufs_and_sems[:NBUF]
    gsem = bufs_and_sems[NBUF : 2 * NBUF]
    ssem = bufs_and_sems[2 * NBUF : 3 * NBUF]
    psem = bufs_and_sems[3 * NBUF]

    wid = lax.axis_index("s") * NC + lax.axis_index("c")
    l0 = wid * LPW

    pe_cp = pltpu.async_copy(pe_hbm.at[pl.ds(l0, LPW)], pe_v, psem)
    for b in range(BATCH):
        pltpu.sync_copy(
            idx_hbm.at[pl.ds(b * SEQ + l0, LPW)], idx_v.at[pl.ds(b * LPW, LPW)]
        )

    chunks = [(b, q) for b in range(BATCH) for q in range(QPB)]
    n = len(chunks)

    def start_gather(c):
        b, q = chunks[c]
        buf = c % NBUF
        return pltpu.async_copy(
            table_hbm.at[idx_v.at[pl.ds(b * LPW + q * CHUNK, CHUNK)]],
            rows[buf],
            gsem[buf],
        )

    pending_g = [None] * NBUF
    pending_s = [None] * NBUF
    for k in range(min(2, n)):
        pending_g[k % NBUF] = start_gather(k)
    pe_cp.wait()

    for c in range(n):
        cur = c % NBUF
        b, q = chunks[c]
        if c + 2 < n:
            nb = (c + 2) % NBUF
            if pending_s[nb] is not None:
                pending_s[nb].wait()
                pending_s[nb] = None
            pending_g[nb] = start_gather(c + 2)
        pending_g[cur].wait()
        pending_g[cur] = None

        def row_body(i, _, cur=cur, q=q):
            for j in range(VECS):
                sl = pl.ds(j * LANES, LANES)
                plsc.addupdate(rows[cur].at[i, sl], pe_v[q * CHUNK + i, sl])
            return _

        lax.fori_loop(0, CHUNK, row_body, 0)
        if pending_s[cur] is not None:
            pending_s[cur].wait()
        pending_s[cur] = pltpu.async_copy(
            rows[cur],
            out_hbm.at[pl.ds(b * SEQ + l0 + q * CHUNK, CHUNK)],
            ssem[cur],
        )
    for s in pending_s:
        if s is not None:
            s.wait()


def kernel(x, table):
    out = _emb(table, x.reshape(-1), jnp.asarray(_PE))
    return out.reshape(BATCH, SEQ, HID)


# speedup vs baseline: 2.6383x; 1.3886x over previous
import functools
import math

import numpy as np
import jax
import jax.numpy as jnp
from jax import lax
from jax.experimental import pallas as pl
from jax.experimental.pallas import tpu as pltpu
from jax.experimental.pallas import tpu_sc as plsc

SEQ = 2048
HID = 1024
BATCH = 4
ROWS = BATCH * SEQ


def _pe_table():
    position = np.arange(0, SEQ, dtype=np.float32)[:, None]
    div_term = np.exp(
        np.arange(0, HID, 2, dtype=np.float32) * (-math.log(10000.0) / HID)
    )
    pe = np.zeros((SEQ, HID), dtype=np.float32)
    pe[:, 0::2] = np.sin(position * div_term)
    pe[:, 1::2] = np.cos(position * div_term)
    import ml_dtypes

    u16 = pe.astype(ml_dtypes.bfloat16).view(np.uint16)
    u16 = u16.reshape(SEQ, HID // 32, 2, 16)
    words = u16[:, :, 0, :].astype(np.uint32) | (
        u16[:, :, 1, :].astype(np.uint32) << 16
    )
    return words.reshape(-1).view(np.int32)


_PE_WORDS = _pe_table()

_INFO = plsc.get_sparse_core_info()
NC, NS, LANES = _INFO.num_cores, _INFO.num_subcores, _INFO.num_lanes
NW = NC * NS
LPW = SEQ // NW
RPW = BATCH * LPW
CHUNK = 16
QPB = LPW // CHUNK
HBLK = HID // 32

_mesh = plsc.VectorSubcoreMesh(core_axis_name="c", subcore_axis_name="s")


@functools.partial(
    pl.kernel,
    mesh=_mesh,
    out_type=jax.ShapeDtypeStruct((ROWS, HID), jnp.float32),
    scratch_types=[
        pltpu.VMEM((RPW,), jnp.int32),
        pltpu.VMEM((LPW * HID // 2,), jnp.int32),
    ]
    + [pltpu.VMEM((CHUNK, HID), jnp.float32) for _ in range(QPB)]
    + [pltpu.SemaphoreType.DMA for _ in range(2 * QPB + 1)],
)
def _emb(table_hbm, idx_hbm, pe_hbm, out_hbm, idx_v, pe_v, *bufs_and_sems):
    rows = bufs_and_sems[:QPB]
    gsem = bufs_and_sems[QPB : 2 * QPB]
    ssem = bufs_and_sems[2 * QPB : 3 * QPB]
    psem = bufs_and_sems[3 * QPB]

    wid = lax.axis_index("s") * NC + lax.axis_index("c")
    l0 = wid * LPW

    pe_cp = pltpu.async_copy(
        pe_hbm.at[pl.ds(l0 * (HID // 2), LPW * (HID // 2))], pe_v, psem
    )
    pltpu.sync_copy(idx_hbm.at[pl.ds(wid * RPW, RPW)], idx_v)

    def start_gather(b, q):
        return pltpu.async_copy(
            table_hbm.at[idx_v.at[pl.ds(b * LPW + q * CHUNK, CHUNK)]],
            rows[q],
            gsem[q],
        )

    def wait_gather(q):
        pltpu.make_async_copy(
            table_hbm.at[idx_v.at[pl.ds(0, CHUNK)]], rows[q], gsem[q]
        ).wait()

    def wait_store(q):
        pltpu.make_async_copy(rows[q], out_hbm.at[pl.ds(0, CHUNK)], ssem[q]).wait()

    start_gather(0, 0)
    start_gather(0, 1)
    pe_cp.wait()

    def batch_body(b, carry):
        for q in range(QPB):
            q2 = (q + 2) % QPB
            if q < 2:
                @pl.when(b >= 1)
                def _():
                    wait_store(q2)

                start_gather(b, q + 2)
            else:
                @pl.when(b < BATCH - 1)
                def _():
                    wait_store(q2)
                    start_gather(b + 1, q - 2)

            wait_gather(q)

            @plsc.parallel_loop(0, CHUNK, unroll=2)
            def row_body(i, q=q):
                pe_base = (q * CHUNK + i) * (HID // 2)
                for j in range(HBLK):
                    w = pe_v[pl.ds(pe_base + j * LANES, LANES)]
                    lo = lax.bitcast_convert_type(lax.shift_left(w, 16), jnp.float32)
                    hi = lax.bitcast_convert_type(
                        lax.bitwise_and(w, jnp.int32(-65536)), jnp.float32
                    )
                    plsc.addupdate(rows[q].at[i, pl.ds(j * 32, LANES)], lo)
                    plsc.addupdate(rows[q].at[i, pl.ds(j * 32 + 16, LANES)], hi)

            pltpu.async_copy(
                rows[q],
                out_hbm.at[pl.ds(b * SEQ + l0 + q * CHUNK, CHUNK)],
                ssem[q],
            )
        return carry

    lax.fori_loop(0, BATCH, batch_body, 0)
    for q in range(QPB):
        wait_store(q)


def kernel(x, table):
    x_perm = x.reshape(BATCH, NW, LPW).transpose(1, 0, 2).reshape(-1)
    pe_w = jnp.asarray(_PE_WORDS)
    out = _emb(table, x_perm, pe_w)
    return out.reshape(BATCH, SEQ, HID)


# speedup vs baseline: 2.6493x; 1.0042x over previous
import functools
import math

import numpy as np
import jax
import jax.numpy as jnp
from jax import lax
from jax.experimental import pallas as pl
from jax.experimental.pallas import tpu as pltpu
from jax.experimental.pallas import tpu_sc as plsc

SEQ = 2048
HID = 1024
BATCH = 4
ROWS = BATCH * SEQ


def _pe_table():
    position = np.arange(0, SEQ, dtype=np.float32)[:, None]
    div_term = np.exp(
        np.arange(0, HID, 2, dtype=np.float32) * (-math.log(10000.0) / HID)
    )
    pe = np.zeros((SEQ, HID), dtype=np.float32)
    pe[:, 0::2] = np.sin(position * div_term)
    pe[:, 1::2] = np.cos(position * div_term)
    import ml_dtypes

    u16 = pe.astype(ml_dtypes.bfloat16).view(np.uint16)
    u16 = u16.reshape(SEQ, HID // 32, 2, 16)
    words = u16[:, :, 0, :].astype(np.uint32) | (
        u16[:, :, 1, :].astype(np.uint32) << 16
    )
    return words.reshape(-1).view(np.int32)


_PE_WORDS = _pe_table()

_INFO = plsc.get_sparse_core_info()
NC, NS, LANES = _INFO.num_cores, _INFO.num_subcores, _INFO.num_lanes
NW = NC * NS
LPW = SEQ // NW
RPW = BATCH * LPW
CHUNK = 16
QPB = LPW // CHUNK
HBLK = HID // 32

_mesh = plsc.VectorSubcoreMesh(core_axis_name="c", subcore_axis_name="s")


@functools.partial(
    pl.kernel,
    mesh=_mesh,
    out_type=jax.ShapeDtypeStruct((ROWS, HID), jnp.float32),
    scratch_types=[
        pltpu.VMEM((RPW,), jnp.int32),
        pltpu.VMEM((LPW * HID // 2,), jnp.int32),
    ]
    + [pltpu.VMEM((CHUNK, HID), jnp.float32) for _ in range(QPB)]
    + [pltpu.SemaphoreType.DMA for _ in range(2 * QPB + 2)],
)
def _emb(table_hbm, idx_hbm, pe_hbm, out_hbm, idx_v, pe_v, *bufs_and_sems):
    rows = bufs_and_sems[:QPB]
    gsem = bufs_and_sems[QPB : 2 * QPB]
    ssem = bufs_and_sems[2 * QPB : 3 * QPB]
    psem = bufs_and_sems[3 * QPB]
    isem = bufs_and_sems[3 * QPB + 1]

    wid = lax.axis_index("s") * NC + lax.axis_index("c")
    l0 = wid * LPW

    pe_cp = pltpu.async_copy(
        pe_hbm.at[pl.ds(l0 * (HID // 2), LPW * (HID // 2))], pe_v, psem
    )
    idx_cps = [
        pltpu.async_copy(
            idx_hbm.at[pl.ds(b * SEQ + l0, LPW)],
            idx_v.at[pl.ds(b * LPW, LPW)],
            isem,
        )
        for b in range(BATCH)
    ]
    for cp in idx_cps:
        cp.wait()

    def start_gather(b, q):
        return pltpu.async_copy(
            table_hbm.at[idx_v.at[pl.ds(b * LPW + q * CHUNK, CHUNK)]],
            rows[q],
            gsem[q],
        )

    def wait_gather(q):
        pltpu.make_async_copy(
            table_hbm.at[idx_v.at[pl.ds(0, CHUNK)]], rows[q], gsem[q]
        ).wait()

    def wait_store(q):
        pltpu.make_async_copy(rows[q], out_hbm.at[pl.ds(0, CHUNK)], ssem[q]).wait()

    start_gather(0, 0)
    start_gather(0, 1)
    pe_cp.wait()

    def batch_body(b, carry):
        for q in range(QPB):
            q2 = (q + 2) % QPB
            if q < 2:
                @pl.when(b >= 1)
                def _():
                    wait_store(q2)

                start_gather(b, q + 2)
            else:
                @pl.when(b < BATCH - 1)
                def _():
                    wait_store(q2)
                    start_gather(b + 1, q - 2)

            wait_gather(q)

            @plsc.parallel_loop(0, CHUNK, unroll=2)
            def row_body(i, q=q):
                pe_base = (q * CHUNK + i) * (HID // 2)
                for j in range(HBLK):
                    w = pe_v[pl.ds(pe_base + j * LANES, LANES)]
                    lo = lax.bitcast_convert_type(lax.shift_left(w, 16), jnp.float32)
                    hi = lax.bitcast_convert_type(
                        lax.bitwise_and(w, jnp.int32(-65536)), jnp.float32
                    )
                    plsc.addupdate(rows[q].at[i, pl.ds(j * 32, LANES)], lo)
                    plsc.addupdate(rows[q].at[i, pl.ds(j * 32 + 16, LANES)], hi)

            pltpu.async_copy(
                rows[q],
                out_hbm.at[pl.ds(b * SEQ + l0 + q * CHUNK, CHUNK)],
                ssem[q],
            )
        return carry

    lax.fori_loop(0, BATCH, batch_body, 0)
    for q in range(QPB):
        wait_store(q)


def kernel(x, table):
    pe_w = jnp.asarray(_PE_WORDS)
    out = _emb(table, x.reshape(-1), pe_w)
    return out.reshape(BATCH, SEQ, HID)


# speedup vs baseline: 2.8297x; 1.0681x over previous
import functools
import math

import numpy as np
import jax
import jax.numpy as jnp
from jax import lax
from jax.experimental import pallas as pl
from jax.experimental.pallas import tpu as pltpu
from jax.experimental.pallas import tpu_sc as plsc

SEQ = 2048
HID = 1024
BATCH = 4
ROWS = BATCH * SEQ


def _pe_table():
    position = np.arange(0, SEQ, dtype=np.float32)[:, None]
    div_term = np.exp(
        np.arange(0, HID, 2, dtype=np.float32) * (-math.log(10000.0) / HID)
    )
    pe = np.zeros((SEQ, HID), dtype=np.float32)
    pe[:, 0::2] = np.sin(position * div_term)
    pe[:, 1::2] = np.cos(position * div_term)
    import ml_dtypes

    u16 = pe.astype(ml_dtypes.bfloat16).view(np.uint16)
    u16 = u16.reshape(SEQ, HID // 32, 2, 16)
    words = u16[:, :, 0, :].astype(np.uint32) | (
        u16[:, :, 1, :].astype(np.uint32) << 16
    )
    return words.reshape(-1).view(np.int32)


_PE_WORDS = _pe_table()

_INFO = plsc.get_sparse_core_info()
NC, NS, LANES = _INFO.num_cores, _INFO.num_subcores, _INFO.num_lanes
NW = NC * NS
LPW = SEQ // NW
RPW = BATCH * LPW
CHUNK = 16
QPB = LPW // CHUNK
HBLK = HID // 32

_mesh = plsc.VectorSubcoreMesh(core_axis_name="c", subcore_axis_name="s")


@functools.partial(
    pl.kernel,
    mesh=_mesh,
    out_type=jax.ShapeDtypeStruct((ROWS, HID), jnp.float32),
    scratch_types=[
        pltpu.VMEM((RPW,), jnp.int32),
        pltpu.VMEM((LPW * HID // 2,), jnp.int32),
    ]
    + [pltpu.VMEM((CHUNK, HID), jnp.float32) for _ in range(QPB)]
    + [pltpu.SemaphoreType.DMA for _ in range(2 * QPB + 2)],
)
def _emb(table_hbm, idx_hbm, pe_hbm, out_hbm, idx_v, pe_v, *bufs_and_sems):
    rows = bufs_and_sems[:QPB]
    gsem = bufs_and_sems[QPB : 2 * QPB]
    ssem = bufs_and_sems[2 * QPB : 3 * QPB]
    psem = bufs_and_sems[3 * QPB]
    isem = bufs_and_sems[3 * QPB + 1]

    wid = lax.axis_index("s") * NC + lax.axis_index("c")
    l0 = wid * LPW

    pe_cp = pltpu.async_copy(
        pe_hbm.at[pl.ds(l0 * (HID // 2), LPW * (HID // 2))], pe_v, psem
    )
    idx_cps = [
        pltpu.async_copy(
            idx_hbm.at[pl.ds(b * SEQ + l0, LPW)],
            idx_v.at[pl.ds(b * LPW, LPW)],
            isem,
        )
        for b in range(BATCH)
    ]
    for cp in idx_cps:
        cp.wait()

    def start_gather(b, q):
        return pltpu.async_copy(
            table_hbm.at[idx_v.at[pl.ds(b * LPW + q * CHUNK, CHUNK)]],
            rows[q],
            gsem[q],
        )

    def wait_gather(q):
        pltpu.make_async_copy(
            table_hbm.at[idx_v.at[pl.ds(0, CHUNK)]], rows[q], gsem[q]
        ).wait()

    def wait_store(q):
        pltpu.make_async_copy(rows[q], out_hbm.at[pl.ds(0, CHUNK)], ssem[q]).wait()

    start_gather(0, 0)
    start_gather(0, 1)
    pe_cp.wait()

    def batch_body(b, carry):
        for q in range(QPB):
            q2 = (q + 2) % QPB
            if q < 2:
                @pl.when(b >= 1)
                def _():
                    wait_store(q2)

                start_gather(b, q + 2)
            else:
                @pl.when(b < BATCH - 1)
                def _():
                    wait_store(q2)
                    start_gather(b + 1, q - 2)

            wait_gather(q)

            @plsc.parallel_loop(0, CHUNK, unroll=4)
            def row_body(i, q=q):
                pe_base = (q * CHUNK + i) * (HID // 2)
                for j in range(HBLK):
                    w = pe_v[pl.ds(pe_base + j * LANES, LANES)]
                    lo = lax.bitcast_convert_type(lax.shift_left(w, 16), jnp.float32)
                    hi = lax.bitcast_convert_type(
                        lax.bitwise_and(w, jnp.int32(-65536)), jnp.float32
                    )
                    plsc.addupdate(rows[q].at[i, pl.ds(j * 32, LANES)], lo)
                    plsc.addupdate(rows[q].at[i, pl.ds(j * 32 + 16, LANES)], hi)

            pltpu.async_copy(
                rows[q],
                out_hbm.at[pl.ds(b * SEQ + l0 + q * CHUNK, CHUNK)],
                ssem[q],
            )
        return carry

    lax.fori_loop(0, BATCH, batch_body, 0)
    for q in range(QPB):
        wait_store(q)


def kernel(x, table):
    pe_w = jnp.asarray(_PE_WORDS)
    out = _emb(table, x.reshape(-1), pe_w)
    return out.reshape(BATCH, SEQ, HID)
